# baseline (speedup 1.0000x reference)
import functools

import jax
import jax.numpy as jnp
from jax import lax
from jax.experimental import pallas as pl
from jax.experimental.pallas import tpu as pltpu
from jax.experimental.pallas import tpu_sc as plsc

N = 1048576
D_FEAT = 64
M_MAG = 256
N_DIR = 16

BLOCK = 8192
HALF = BLOCK // 2
GRID = N // BLOCK

SC_CORES = 2
SC_SUBCORES = 16
SC_LANES = 16
NW = SC_CORES * SC_SUBCORES
CHUNK = N // NW


def _tc_digitize_kernel(x_ref, mean_ref, std_ref, a_ref, at_ref, dirs_ref,
                        counts_ref, flat_ref, table_ref):
    b = x_ref.shape[0]
    inv_std = 1.0 / std_ref[...]
    v = (x_ref[...] - mean_ref[...]) * inv_std
    vt = jnp.transpose(v)

    def stream(vth):
        sq_t = jnp.sum(vth * vth, axis=0, keepdims=True)
        mag_t = jnp.sqrt(sq_t)
        unit_t = vth * (1.0 / (mag_t + 1e-12))
        sims_t = lax.dot_general(dirs_ref[...], unit_t,
                                 (((1,), (0,)), ((), ())),
                                 preferred_element_type=jnp.float32)
        si = lax.bitcast_convert_type(sims_t, jnp.int32)
        key = (si ^ ((si >> 31) & jnp.int32(0x7FFFFFFF))) | jnp.int32(15)
        io16 = lax.broadcasted_iota(jnp.int32, (N_DIR, b), 0)
        key = key - io16
        dir_t = 15 - (jnp.max(key, axis=0, keepdims=True) & jnp.int32(15))

        coarse = a_ref[...][:, 0:1]
        cmat = (coarse <= mag_t).astype(jnp.float32)
        cc = jnp.sum(cmat, axis=0, keepdims=True)
        onehot_c = cmat - jnp.concatenate(
            [cmat[1:, :], jnp.zeros((1, b), dtype=jnp.float32)], axis=0)
        fine16 = lax.dot_general(at_ref[...], onehot_c,
                                 (((1,), (0,)), ((), ())),
                                 precision=lax.Precision.HIGHEST,
                                 preferred_element_type=jnp.float32)
        fc = jnp.sum((fine16 <= mag_t).astype(jnp.float32), axis=0,
                     keepdims=True)
        mag_idx = jnp.clip(
            (16.0 * (cc - 1.0) + fc - 1.0).astype(jnp.int32), 0, M_MAG - 1)
        return dir_t * M_MAG + mag_idx

    flat_even = stream(vt[:D_FEAT, :])
    flat_odd = stream(vt[D_FEAT:, :])
    flat_ref[...] = jnp.concatenate(
        [flat_even, flat_odd], axis=0).reshape(1, 2, b)

    tot = jnp.maximum(jnp.int32(1), jnp.sum(counts_ref[...]))
    table_ref[...] = counts_ref[...].astype(jnp.float32) / tot.astype(jnp.float32)


def _tc_digitize(vectors, mean, std, edges, dirs, counts, interpret=False):
    a = edges.reshape(16, 16)
    at = a.T
    mean2 = jnp.concatenate([mean, mean]).reshape(1, 2 * D_FEAT)
    std2 = jnp.concatenate([std, std]).reshape(1, 2 * D_FEAT)
    xw = vectors.reshape(N // 2, 2 * D_FEAT)
    flat, table = pl.pallas_call(
        _tc_digitize_kernel,
        grid=(GRID,),
        in_specs=[
            pl.BlockSpec((HALF, 2 * D_FEAT), lambda i: (i, 0)),
            pl.BlockSpec((1, 2 * D_FEAT), lambda i: (0, 0)),
            pl.BlockSpec((1, 2 * D_FEAT), lambda i: (0, 0)),
            pl.BlockSpec((16, 16), lambda i: (0, 0)),
            pl.BlockSpec((16, 16), lambda i: (0, 0)),
            pl.BlockSpec((N_DIR, D_FEAT), lambda i: (0, 0)),
            pl.BlockSpec((N_DIR, M_MAG), lambda i: (0, 0)),
        ],
        out_specs=[
            pl.BlockSpec((1, 2, HALF), lambda i: (i, 0, 0)),
            pl.BlockSpec((N_DIR, M_MAG), lambda i: (0, 0)),
        ],
        out_shape=[
            jax.ShapeDtypeStruct((GRID, 2, HALF), jnp.int32),
            jax.ShapeDtypeStruct((N_DIR, M_MAG), jnp.float32),
        ],
        compiler_params=pltpu.CompilerParams(
            dimension_semantics=("arbitrary",)),
        interpret=interpret,
    )(xw, mean2, std2, a, at, dirs, counts)
    flat = jnp.transpose(flat, (0, 2, 1)).reshape(N)
    return flat, table


def _sc_gather(table_flat, flat_idx):
    mesh = plsc.VectorSubcoreMesh(core_axis_name="c", subcore_axis_name="s")

    @functools.partial(
        pl.kernel, mesh=mesh,
        out_type=jax.ShapeDtypeStruct((N,), jnp.float32),
        compiler_params=pltpu.CompilerParams(needs_layout_passes=False),
        scratch_types=[
            pltpu.VMEM((N_DIR * M_MAG,), jnp.float32),
            pltpu.VMEM((CHUNK,), jnp.int32),
            pltpu.VMEM((CHUNK,), jnp.float32),
        ],
    )
    def gather_kernel(table_hbm, idx_hbm, out_hbm, table_v, idx_v, out_v):
        wid = lax.axis_index("s") * SC_CORES + lax.axis_index("c")
        base = wid * CHUNK
        pltpu.sync_copy(table_hbm, table_v)
        pltpu.sync_copy(idx_hbm.at[pl.ds(base, CHUNK)], idx_v)

        def body(i, _):
            off = i * SC_LANES
            iv = idx_v[pl.ds(off, SC_LANES)]
            out_v[pl.ds(off, SC_LANES)] = plsc.load_gather(table_v, [iv])
            return 0

        lax.fori_loop(0, CHUNK // SC_LANES, body, 0, unroll=8)
        pltpu.sync_copy(out_v, out_hbm.at[pl.ds(base, CHUNK)])

    return gather_kernel(table_flat, flat_idx)


def kernel(vectors, mean, std, reference_magnitudes, reference_directions,
           counts):
    flat, table = _tc_digitize(vectors, mean, std, reference_magnitudes,
                               reference_directions, counts)
    probs = _sc_gather(table.reshape(N_DIR * M_MAG), flat)
    return probs


# speedup vs baseline: 1.7486x; 1.7486x over previous
import functools

import jax
import jax.numpy as jnp
from jax import lax
from jax.experimental import pallas as pl
from jax.experimental.pallas import tpu as pltpu
from jax.experimental.pallas import tpu_sc as plsc

N = 1048576
D_FEAT = 64
M_MAG = 256
N_DIR = 16

BLOCK = 8192
HALF = BLOCK // 2
GRID = N // BLOCK

SC_CORES = 2
SC_SUBCORES = 16
SC_LANES = 16
NW = SC_CORES * SC_SUBCORES
CHUNK = N // NW
PCHUNK = CHUNK // 2


def _tc_digitize_kernel(x_ref, mean_ref, std_ref, a_ref, at_ref, dirs_ref,
                        counts_ref, flat_ref, table_ref):
    b = x_ref.shape[0]
    inv_std = 1.0 / std_ref[...]
    v = (x_ref[...] - mean_ref[...]) * inv_std

    vt = jnp.transpose(v)
    sq_t = jnp.sum(vt * vt, axis=0, keepdims=True)
    mag_t = jnp.sqrt(sq_t)

    unit_t = vt * (1.0 / (mag_t + 1e-12))
    sims_t = lax.dot_general(dirs_ref[...], unit_t, (((1,), (0,)), ((), ())),
                             preferred_element_type=jnp.float32)
    si = lax.bitcast_convert_type(sims_t, jnp.int32)
    key = (si ^ ((si >> 31) & jnp.int32(0x7FFFFFFF))) | jnp.int32(15)
    io16 = lax.broadcasted_iota(jnp.int32, (N_DIR, b), 0)
    key = key - io16
    dir_t = 15 - (jnp.max(key, axis=0, keepdims=True) & jnp.int32(15))

    coarse = a_ref[...][:, 0:1]
    cmat = (coarse <= mag_t).astype(jnp.float32)
    cc = jnp.sum(cmat, axis=0, keepdims=True)
    onehot_c = cmat - jnp.concatenate(
        [cmat[1:, :], jnp.zeros((1, b), dtype=jnp.float32)], axis=0)
    fine16 = lax.dot_general(at_ref[...], onehot_c, (((1,), (0,)), ((), ())),
                             precision=lax.Precision.HIGHEST,
                             preferred_element_type=jnp.float32)
    fc = jnp.sum((fine16 <= mag_t).astype(jnp.float32), axis=0, keepdims=True)
    mag_idx = jnp.clip(
        (16.0 * (cc - 1.0) + fc - 1.0).astype(jnp.int32), 0, M_MAG - 1)

    flat = dir_t * M_MAG + mag_idx
    h = b // 2
    packed = flat[:, :h] | (flat[:, h:] << 16)
    flat_ref[...] = packed.reshape(1, 1, h)

    tot = jnp.maximum(jnp.int32(1), jnp.sum(counts_ref[...]))
    table_ref[...] = counts_ref[...].astype(jnp.float32) / tot.astype(jnp.float32)


def _tc_digitize(vectors, mean, std, edges, dirs, counts, interpret=False):
    a = edges.reshape(16, 16)
    at = a.T
    return pl.pallas_call(
        _tc_digitize_kernel,
        grid=(GRID,),
        in_specs=[
            pl.BlockSpec((BLOCK, D_FEAT), lambda i: (i, 0)),
            pl.BlockSpec((1, D_FEAT), lambda i: (0, 0)),
            pl.BlockSpec((1, D_FEAT), lambda i: (0, 0)),
            pl.BlockSpec((16, 16), lambda i: (0, 0)),
            pl.BlockSpec((16, 16), lambda i: (0, 0)),
            pl.BlockSpec((N_DIR, D_FEAT), lambda i: (0, 0)),
            pl.BlockSpec((N_DIR, M_MAG), lambda i: (0, 0)),
        ],
        out_specs=[
            pl.BlockSpec((1, 1, HALF), lambda i: (i, 0, 0)),
            pl.BlockSpec((N_DIR, M_MAG), lambda i: (0, 0)),
        ],
        out_shape=[
            jax.ShapeDtypeStruct((GRID, 1, HALF), jnp.int32),
            jax.ShapeDtypeStruct((N_DIR, M_MAG), jnp.float32),
        ],
        compiler_params=pltpu.CompilerParams(
            dimension_semantics=("arbitrary",)),
        interpret=interpret,
    )(vectors, mean.reshape(1, D_FEAT), std.reshape(1, D_FEAT), a, at,
      dirs, counts)


def _sc_gather(table_flat, packed_idx):
    mesh = plsc.VectorSubcoreMesh(core_axis_name="c", subcore_axis_name="s")
    bpw = PCHUNK // HALF

    @functools.partial(
        pl.kernel, mesh=mesh,
        out_type=jax.ShapeDtypeStruct((N,), jnp.float32),
        compiler_params=pltpu.CompilerParams(needs_layout_passes=False),
        scratch_types=[
            pltpu.VMEM((N_DIR * M_MAG,), jnp.float32),
            pltpu.VMEM((PCHUNK,), jnp.int32),
            pltpu.VMEM((CHUNK,), jnp.float32),
        ],
    )
    def gather_kernel(table_hbm, idx_hbm, out_hbm, table_v, idx_v, out_v):
        wid = lax.axis_index("s") * SC_CORES + lax.axis_index("c")
        pltpu.sync_copy(table_hbm, table_v)
        pltpu.sync_copy(idx_hbm.at[pl.ds(wid * PCHUNK, PCHUNK)], idx_v)

        for t in range(bpw):
            def body(i, _, t=t):
                off = t * HALF + i * SC_LANES
                w = idx_v[pl.ds(off, SC_LANES)]
                lo = w & 0xFFFF
                hi = w >> 16
                o = t * BLOCK + i * SC_LANES
                out_v[pl.ds(o, SC_LANES)] = plsc.load_gather(table_v, [lo])
                out_v[pl.ds(o + HALF, SC_LANES)] = \
                    plsc.load_gather(table_v, [hi])
                return 0

            lax.fori_loop(0, HALF // SC_LANES, body, 0, unroll=8)
        pltpu.sync_copy(out_v, out_hbm.at[pl.ds(wid * CHUNK, CHUNK)])

    return gather_kernel(table_flat, packed_idx)


def kernel(vectors, mean, std, reference_magnitudes, reference_directions,
           counts):
    flat, table = _tc_digitize(vectors, mean, std, reference_magnitudes,
                               reference_directions, counts)
    probs = _sc_gather(table.reshape(N_DIR * M_MAG), flat.reshape(N // 2))
    return probs


# speedup vs baseline: 4.7854x; 2.7368x over previous
import functools

import jax
import jax.numpy as jnp
from jax import lax
from jax.experimental import pallas as pl
from jax.experimental.pallas import tpu as pltpu
from jax.experimental.pallas import tpu_sc as plsc

N = 1048576
D_FEAT = 64
M_MAG = 256
N_DIR = 16

BLOCK = 8192
HALF = BLOCK // 2
GRID = N // BLOCK

SC_CORES = 2
SC_SUBCORES = 16
SC_LANES = 16
NW = SC_CORES * SC_SUBCORES
CHUNK = N // NW
PCHUNK = CHUNK // 2


def _tc_digitize_kernel(x_ref, mean_ref, std_ref, a_ref, at_ref, dirs_ref,
                        counts_ref, flat_ref, table_ref):
    b = x_ref.shape[1]
    inv_std = 1.0 / std_ref[...]
    vt = (x_ref[...] - mean_ref[...]) * inv_std
    sq_t = jnp.sum(vt * vt, axis=0, keepdims=True)
    mag_t = jnp.sqrt(sq_t)

    unit_t = vt * (1.0 / (mag_t + 1e-12))
    sims_t = lax.dot_general(dirs_ref[...], unit_t, (((1,), (0,)), ((), ())),
                             preferred_element_type=jnp.float32)
    si = lax.bitcast_convert_type(sims_t, jnp.int32)
    key = (si ^ ((si >> 31) & jnp.int32(0x7FFFFFFF))) | jnp.int32(15)
    io16 = lax.broadcasted_iota(jnp.int32, (N_DIR, b), 0)
    key = key - io16
    dir_t = 15 - (jnp.max(key, axis=0, keepdims=True) & jnp.int32(15))

    coarse = a_ref[...][:, 0:1]
    cmat = (coarse <= mag_t).astype(jnp.float32)
    cc = jnp.sum(cmat, axis=0, keepdims=True)
    onehot_c = cmat - jnp.concatenate(
        [cmat[1:, :], jnp.zeros((1, b), dtype=jnp.float32)], axis=0)
    fine16 = lax.dot_general(at_ref[...], onehot_c, (((1,), (0,)), ((), ())),
                             precision=lax.Precision.HIGHEST,
                             preferred_element_type=jnp.float32)
    fc = jnp.sum((fine16 <= mag_t).astype(jnp.float32), axis=0, keepdims=True)
    mag_idx = jnp.clip(
        (16.0 * (cc - 1.0) + fc - 1.0).astype(jnp.int32), 0, M_MAG - 1)

    flat = dir_t * M_MAG + mag_idx
    h = b // 2
    packed = flat[:, :h] | (flat[:, h:] << 16)
    flat_ref[...] = packed.reshape(1, 1, h)

    tot = jnp.maximum(jnp.int32(1), jnp.sum(counts_ref[...]))
    table_ref[...] = counts_ref[...].astype(jnp.float32) / tot.astype(jnp.float32)


def _tc_digitize(vectors, mean, std, edges, dirs, counts, interpret=False):
    a = edges.reshape(16, 16)
    at = a.T
    return pl.pallas_call(
        _tc_digitize_kernel,
        grid=(GRID,),
        in_specs=[
            pl.BlockSpec((D_FEAT, BLOCK), lambda i: (0, i)),
            pl.BlockSpec((D_FEAT, 1), lambda i: (0, 0)),
            pl.BlockSpec((D_FEAT, 1), lambda i: (0, 0)),
            pl.BlockSpec((16, 16), lambda i: (0, 0)),
            pl.BlockSpec((16, 16), lambda i: (0, 0)),
            pl.BlockSpec((N_DIR, D_FEAT), lambda i: (0, 0)),
            pl.BlockSpec((N_DIR, M_MAG), lambda i: (0, 0)),
        ],
        out_specs=[
            pl.BlockSpec((1, 1, HALF), lambda i: (i, 0, 0)),
            pl.BlockSpec((N_DIR, M_MAG), lambda i: (0, 0)),
        ],
        out_shape=[
            jax.ShapeDtypeStruct((GRID, 1, HALF), jnp.int32),
            jax.ShapeDtypeStruct((N_DIR, M_MAG), jnp.float32),
        ],
        compiler_params=pltpu.CompilerParams(
            dimension_semantics=("arbitrary",)),
        interpret=interpret,
    )(vectors.T, mean.reshape(D_FEAT, 1), std.reshape(D_FEAT, 1), a, at,
      dirs, counts)


def _sc_gather(table_flat, packed_idx):
    mesh = plsc.VectorSubcoreMesh(core_axis_name="c", subcore_axis_name="s")
    bpw = PCHUNK // HALF

    @functools.partial(
        pl.kernel, mesh=mesh,
        out_type=jax.ShapeDtypeStruct((N,), jnp.float32),
        compiler_params=pltpu.CompilerParams(needs_layout_passes=False),
        scratch_types=[
            pltpu.VMEM((N_DIR * M_MAG,), jnp.float32),
            pltpu.VMEM((PCHUNK,), jnp.int32),
            pltpu.VMEM((CHUNK,), jnp.float32),
        ],
    )
    def gather_kernel(table_hbm, idx_hbm, out_hbm, table_v, idx_v, out_v):
        wid = lax.axis_index("s") * SC_CORES + lax.axis_index("c")
        pltpu.sync_copy(table_hbm, table_v)
        pltpu.sync_copy(idx_hbm.at[pl.ds(wid * PCHUNK, PCHUNK)], idx_v)

        for t in range(bpw):
            def body(i, _, t=t):
                off = t * HALF + i * SC_LANES
                w = idx_v[pl.ds(off, SC_LANES)]
                lo = w & 0xFFFF
                hi = w >> 16
                o = t * BLOCK + i * SC_LANES
                out_v[pl.ds(o, SC_LANES)] = plsc.load_gather(table_v, [lo])
                out_v[pl.ds(o + HALF, SC_LANES)] = \
                    plsc.load_gather(table_v, [hi])
                return 0

            lax.fori_loop(0, HALF // SC_LANES, body, 0, unroll=8)
        pltpu.sync_copy(out_v, out_hbm.at[pl.ds(wid * CHUNK, CHUNK)])

    return gather_kernel(table_flat, packed_idx)


def kernel(vectors, mean, std, reference_magnitudes, reference_directions,
           counts):
    flat, table = _tc_digitize(vectors, mean, std, reference_magnitudes,
                               reference_directions, counts)
    probs = _sc_gather(table.reshape(N_DIR * M_MAG), flat.reshape(N // 2))
    return probs


# speedup vs baseline: 5.0793x; 1.0614x over previous
import functools

import jax
import jax.numpy as jnp
from jax import lax
from jax.experimental import pallas as pl
from jax.experimental.pallas import tpu as pltpu
from jax.experimental.pallas import tpu_sc as plsc

N = 1048576
D_FEAT = 64
M_MAG = 256
N_DIR = 16

BLOCK = 16384
HALF = BLOCK // 2
GRID = N // BLOCK

SC_CORES = 2
SC_SUBCORES = 16
SC_LANES = 16
NW = SC_CORES * SC_SUBCORES
CHUNK = N // NW
PCHUNK = CHUNK // 2


def _tc_digitize_kernel(x_ref, mean_ref, std_ref, a_ref, at_ref, dirs_ref,
                        counts_ref, flat_ref, table_ref):
    b = x_ref.shape[1]
    inv_std = 1.0 / std_ref[...]
    vt = (x_ref[...] - mean_ref[...]) * inv_std
    sq_t = jnp.sum(vt * vt, axis=0, keepdims=True)
    mag_t = jnp.sqrt(sq_t)

    unit_t = vt * (1.0 / (mag_t + 1e-12))
    sims_t = lax.dot_general(dirs_ref[...], unit_t, (((1,), (0,)), ((), ())),
                             preferred_element_type=jnp.float32)
    si = lax.bitcast_convert_type(sims_t, jnp.int32)
    key = (si ^ ((si >> 31) & jnp.int32(0x7FFFFFFF))) | jnp.int32(15)
    io16 = lax.broadcasted_iota(jnp.int32, (N_DIR, b), 0)
    key = key - io16
    dir_t = 15 - (jnp.max(key, axis=0, keepdims=True) & jnp.int32(15))

    coarse = a_ref[...][:, 0:1]
    cmat = (coarse <= mag_t).astype(jnp.float32)
    cc = jnp.sum(cmat, axis=0, keepdims=True)
    onehot_c = cmat - jnp.concatenate(
        [cmat[1:, :], jnp.zeros((1, b), dtype=jnp.float32)], axis=0)
    fine16 = lax.dot_general(at_ref[...], onehot_c, (((1,), (0,)), ((), ())),
                             precision=lax.Precision.HIGHEST,
                             preferred_element_type=jnp.float32)
    fc = jnp.sum((fine16 <= mag_t).astype(jnp.float32), axis=0, keepdims=True)
    mag_idx = jnp.clip(
        (16.0 * (cc - 1.0) + fc - 1.0).astype(jnp.int32), 0, M_MAG - 1)

    flat = dir_t * M_MAG + mag_idx
    h = b // 2
    packed = flat[:, :h] | (flat[:, h:] << 16)
    flat_ref[...] = packed.reshape(1, 1, h)

    tot = jnp.maximum(jnp.int32(1), jnp.sum(counts_ref[...]))
    table_ref[...] = counts_ref[...].astype(jnp.float32) / tot.astype(jnp.float32)


def _tc_digitize(vectors, mean, std, edges, dirs, counts, interpret=False):
    a = edges.reshape(16, 16)
    at = a.T
    return pl.pallas_call(
        _tc_digitize_kernel,
        grid=(GRID,),
        in_specs=[
            pl.BlockSpec((D_FEAT, BLOCK), lambda i: (0, i)),
            pl.BlockSpec((D_FEAT, 1), lambda i: (0, 0)),
            pl.BlockSpec((D_FEAT, 1), lambda i: (0, 0)),
            pl.BlockSpec((16, 16), lambda i: (0, 0)),
            pl.BlockSpec((16, 16), lambda i: (0, 0)),
            pl.BlockSpec((N_DIR, D_FEAT), lambda i: (0, 0)),
            pl.BlockSpec((N_DIR, M_MAG), lambda i: (0, 0)),
        ],
        out_specs=[
            pl.BlockSpec((1, 1, HALF), lambda i: (i, 0, 0)),
            pl.BlockSpec((N_DIR, M_MAG), lambda i: (0, 0)),
        ],
        out_shape=[
            jax.ShapeDtypeStruct((GRID, 1, HALF), jnp.int32),
            jax.ShapeDtypeStruct((N_DIR, M_MAG), jnp.float32),
        ],
        compiler_params=pltpu.CompilerParams(
            dimension_semantics=("arbitrary",)),
        interpret=interpret,
    )(vectors.T, mean.reshape(D_FEAT, 1), std.reshape(D_FEAT, 1), a, at,
      dirs, counts)


def _sc_gather(table_flat, packed_idx):
    mesh = plsc.VectorSubcoreMesh(core_axis_name="c", subcore_axis_name="s")
    bpw = PCHUNK // HALF

    @functools.partial(
        pl.kernel, mesh=mesh,
        out_type=jax.ShapeDtypeStruct((N,), jnp.float32),
        compiler_params=pltpu.CompilerParams(needs_layout_passes=False),
        scratch_types=[
            pltpu.VMEM((N_DIR * M_MAG,), jnp.float32),
            pltpu.VMEM((PCHUNK,), jnp.int32),
            pltpu.VMEM((CHUNK,), jnp.float32),
        ],
    )
    def gather_kernel(table_hbm, idx_hbm, out_hbm, table_v, idx_v, out_v):
        wid = lax.axis_index("s") * SC_CORES + lax.axis_index("c")
        pltpu.sync_copy(table_hbm, table_v)
        pltpu.sync_copy(idx_hbm.at[pl.ds(wid * PCHUNK, PCHUNK)], idx_v)

        for t in range(bpw):
            def body(i, _, t=t):
                off = t * HALF + i * SC_LANES
                w = idx_v[pl.ds(off, SC_LANES)]
                lo = w & 0xFFFF
                hi = w >> 16
                o = t * BLOCK + i * SC_LANES
                out_v[pl.ds(o, SC_LANES)] = plsc.load_gather(table_v, [lo])
                out_v[pl.ds(o + HALF, SC_LANES)] = \
                    plsc.load_gather(table_v, [hi])
                return 0

            lax.fori_loop(0, HALF // SC_LANES, body, 0, unroll=8)
        pltpu.sync_copy(out_v, out_hbm.at[pl.ds(wid * CHUNK, CHUNK)])

    return gather_kernel(table_flat, packed_idx)


def kernel(vectors, mean, std, reference_magnitudes, reference_directions,
           counts):
    flat, table = _tc_digitize(vectors, mean, std, reference_magnitudes,
                               reference_directions, counts)
    probs = _sc_gather(table.reshape(N_DIR * M_MAG), flat.reshape(N // 2))
    return probs


# speedup vs baseline: 5.2856x; 1.0406x over previous
import functools

import jax
import jax.numpy as jnp
from jax import lax
from jax.experimental import pallas as pl
from jax.experimental.pallas import tpu as pltpu
from jax.experimental.pallas import tpu_sc as plsc

N = 1048576
D_FEAT = 64
M_MAG = 256
N_DIR = 16

BLOCK = 32768
HALF = BLOCK // 2
GRID = N // BLOCK

SC_CORES = 2
SC_SUBCORES = 16
SC_LANES = 16
NW = SC_CORES * SC_SUBCORES
CHUNK = N // NW
PCHUNK = CHUNK // 2


def _tc_digitize_kernel(x_ref, mean_ref, std_ref, a_ref, at_ref, dirs_ref,
                        counts_ref, flat_ref, table_ref):
    b = x_ref.shape[1]
    inv_std = 1.0 / std_ref[...]
    vt = (x_ref[...] - mean_ref[...]) * inv_std
    sq_t = jnp.sum(vt * vt, axis=0, keepdims=True)
    mag_t = jnp.sqrt(sq_t)

    unit_t = vt * (1.0 / (mag_t + 1e-12))
    sims_t = lax.dot_general(dirs_ref[...], unit_t, (((1,), (0,)), ((), ())),
                             preferred_element_type=jnp.float32)
    si = lax.bitcast_convert_type(sims_t, jnp.int32)
    key = (si ^ ((si >> 31) & jnp.int32(0x7FFFFFFF))) | jnp.int32(15)
    io16 = lax.broadcasted_iota(jnp.int32, (N_DIR, b), 0)
    key = key - io16
    dir_t = 15 - (jnp.max(key, axis=0, keepdims=True) & jnp.int32(15))

    coarse = a_ref[...][:, 0:1]
    cmat = (coarse <= mag_t).astype(jnp.float32)
    cc = jnp.sum(cmat, axis=0, keepdims=True)
    onehot_c = cmat - jnp.concatenate(
        [cmat[1:, :], jnp.zeros((1, b), dtype=jnp.float32)], axis=0)
    fine16 = lax.dot_general(at_ref[...], onehot_c, (((1,), (0,)), ((), ())),
                             precision=lax.Precision.HIGHEST,
                             preferred_element_type=jnp.float32)
    fc = jnp.sum((fine16 <= mag_t).astype(jnp.float32), axis=0, keepdims=True)
    mag_idx = jnp.clip(
        (16.0 * (cc - 1.0) + fc - 1.0).astype(jnp.int32), 0, M_MAG - 1)

    flat = dir_t * M_MAG + mag_idx
    h = b // 2
    packed = flat[:, :h] | (flat[:, h:] << 16)
    flat_ref[...] = packed.reshape(1, 1, h)

    tot = jnp.maximum(jnp.int32(1), jnp.sum(counts_ref[...]))
    table_ref[...] = counts_ref[...].astype(jnp.float32) / tot.astype(jnp.float32)


def _tc_digitize(vectors, mean, std, edges, dirs, counts, interpret=False):
    a = edges.reshape(16, 16)
    at = a.T
    return pl.pallas_call(
        _tc_digitize_kernel,
        grid=(GRID,),
        in_specs=[
            pl.BlockSpec((D_FEAT, BLOCK), lambda i: (0, i)),
            pl.BlockSpec((D_FEAT, 1), lambda i: (0, 0)),
            pl.BlockSpec((D_FEAT, 1), lambda i: (0, 0)),
            pl.BlockSpec((16, 16), lambda i: (0, 0)),
            pl.BlockSpec((16, 16), lambda i: (0, 0)),
            pl.BlockSpec((N_DIR, D_FEAT), lambda i: (0, 0)),
            pl.BlockSpec((N_DIR, M_MAG), lambda i: (0, 0)),
        ],
        out_specs=[
            pl.BlockSpec((1, 1, HALF), lambda i: (i, 0, 0)),
            pl.BlockSpec((N_DIR, M_MAG), lambda i: (0, 0)),
        ],
        out_shape=[
            jax.ShapeDtypeStruct((GRID, 1, HALF), jnp.int32),
            jax.ShapeDtypeStruct((N_DIR, M_MAG), jnp.float32),
        ],
        compiler_params=pltpu.CompilerParams(
            dimension_semantics=("arbitrary",)),
        interpret=interpret,
    )(vectors.T, mean.reshape(D_FEAT, 1), std.reshape(D_FEAT, 1), a, at,
      dirs, counts)


def _sc_gather(table_flat, packed_idx):
    mesh = plsc.VectorSubcoreMesh(core_axis_name="c", subcore_axis_name="s")
    bpw = PCHUNK // HALF

    @functools.partial(
        pl.kernel, mesh=mesh,
        out_type=jax.ShapeDtypeStruct((N,), jnp.float32),
        compiler_params=pltpu.CompilerParams(needs_layout_passes=False),
        scratch_types=[
            pltpu.VMEM((N_DIR * M_MAG,), jnp.float32),
            pltpu.VMEM((PCHUNK,), jnp.int32),
            pltpu.VMEM((CHUNK,), jnp.float32),
        ],
    )
    def gather_kernel(table_hbm, idx_hbm, out_hbm, table_v, idx_v, out_v):
        wid = lax.axis_index("s") * SC_CORES + lax.axis_index("c")
        pltpu.sync_copy(table_hbm, table_v)
        pltpu.sync_copy(idx_hbm.at[pl.ds(wid * PCHUNK, PCHUNK)], idx_v)

        for t in range(bpw):
            def body(i, _, t=t):
                off = t * HALF + i * SC_LANES
                w = idx_v[pl.ds(off, SC_LANES)]
                lo = w & 0xFFFF
                hi = w >> 16
                o = t * BLOCK + i * SC_LANES
                out_v[pl.ds(o, SC_LANES)] = plsc.load_gather(table_v, [lo])
                out_v[pl.ds(o + HALF, SC_LANES)] = \
                    plsc.load_gather(table_v, [hi])
                return 0

            lax.fori_loop(0, HALF // SC_LANES, body, 0, unroll=8)
        pltpu.sync_copy(out_v, out_hbm.at[pl.ds(wid * CHUNK, CHUNK)])

    return gather_kernel(table_flat, packed_idx)


def kernel(vectors, mean, std, reference_magnitudes, reference_directions,
           counts):
    flat, table = _tc_digitize(vectors, mean, std, reference_magnitudes,
                               reference_directions, counts)
    probs = _sc_gather(table.reshape(N_DIR * M_MAG), flat.reshape(N // 2))
    return probs
